# Initial kernel scaffold; baseline (speedup 1.0000x reference)
import jax
import jax.numpy as jnp
from jax.experimental import pallas as pl


def kernel(action_type, x, y, action_type_table, x_table, y_table):
    raise NotImplementedError("write your pallas kernel here")


# speedup vs baseline: 18.6324x; 18.6324x over previous
import functools

import jax
import jax.numpy as jnp
from jax import lax
from jax.experimental import pallas as pl
from jax.experimental.pallas import tpu as pltpu
from jax.experimental.pallas import tpu_sc as plsc

D = 128
N_ACT = 9
N_GRID = 64
ROW = 128
CHUNK = 8


def _build_fused_table(a_tab, x_tab, y_tab):
    def body(a_ref, x_ref, y_ref, o_ref):
        av = a_ref[...]
        xv = x_ref[...]
        yv = y_ref[...]
        o_ref[...] = ((av[:, None, None, :] + xv[None, :, None, :])
                      + yv[None, None, :, :])

    out = pl.pallas_call(
        body,
        out_shape=jax.ShapeDtypeStruct((N_ACT, N_GRID, N_GRID, D),
                                       jnp.float32),
    )(a_tab, x_tab, y_tab)
    return out.reshape(N_ACT * N_GRID * N_GRID, D)


def _sc_lookup(at_idx, x_idx, y_idx, axy):
    nrows = at_idx.shape[0]
    info = plsc.get_sparse_core_info()
    nw = info.num_cores * info.num_subcores
    assert nrows % (nw * CHUNK) == 0, (nrows, nw)
    rows_per_w = nrows // nw
    steps = rows_per_w // CHUNK

    mesh = plsc.VectorSubcoreMesh(core_axis_name="c", subcore_axis_name="s")

    @functools.partial(
        pl.kernel,
        out_type=jax.ShapeDtypeStruct((nrows * ROW, D), jnp.float32),
        mesh=mesh,
        scratch_types=[
            pltpu.VMEM((CHUNK, ROW), jnp.int32),
            pltpu.VMEM((CHUNK, ROW), jnp.int32),
            pltpu.VMEM((CHUNK, ROW), jnp.int32),
            pltpu.VMEM((CHUNK, ROW), jnp.int32),
            pltpu.VMEM((ROW, D), jnp.float32),
            pltpu.SemaphoreType.DMA,
        ],
    )
    def k(at_hbm, x_hbm, y_hbm, axy_hbm, out_hbm,
          ai_v, xi_v, yi_v, idx_v, rows_v, sem):
        wid = lax.axis_index("s") * info.num_cores + lax.axis_index("c")
        row_base = wid * rows_per_w

        @pl.loop(0, steps)
        def _step(i):
            row0 = row_base + i * CHUNK
            pltpu.sync_copy(at_hbm.at[pl.ds(row0, CHUNK)], ai_v)
            pltpu.sync_copy(x_hbm.at[pl.ds(row0, CHUNK)], xi_v)
            pltpu.sync_copy(y_hbm.at[pl.ds(row0, CHUNK)], yi_v)
            for j in range(CHUNK):
                for s in range(ROW // 16):
                    sl = pl.ds(s * 16, 16)
                    a = ai_v[j, sl]
                    xx = xi_v[j, sl]
                    yy = yi_v[j, sl]
                    idx_v[j, sl] = (a << 12) | (xx << 6) | yy
            for j in range(CHUNK):
                pltpu.async_copy(axy_hbm.at[idx_v.at[j]], rows_v, sem).wait()
                pltpu.sync_copy(
                    rows_v, out_hbm.at[pl.ds((row0 + j) * ROW, ROW)])

    return k(at_idx, x_idx, y_idx, axy)


def kernel(action_type, x, y, action_type_table, x_table, y_table):
    B, T = action_type.shape
    n = B * T
    assert n % ROW == 0
    axy = _build_fused_table(action_type_table, x_table, y_table)
    at2 = action_type.reshape(n // ROW, ROW).astype(jnp.int32)
    x2 = x.reshape(n // ROW, ROW).astype(jnp.int32)
    y2 = y.reshape(n // ROW, ROW).astype(jnp.int32)
    out = _sc_lookup(at2, x2, y2, axy)
    return out.reshape(B, T, D)


# speedup vs baseline: 22.0971x; 1.1859x over previous
import functools

import jax
import jax.numpy as jnp
from jax import lax
from jax.experimental import pallas as pl
from jax.experimental.pallas import tpu as pltpu
from jax.experimental.pallas import tpu_sc as plsc

D = 128
N_ACT = 9
N_GRID = 64
ROW = 128
CHUNK = 8


def _build_fused_table(a_tab, x_tab, y_tab):
    def body(a_ref, x_ref, y_ref, o_ref):
        av = a_ref[...]
        xv = x_ref[...]
        yv = y_ref[...]
        o_ref[...] = ((av[:, None, None, :] + xv[None, :, None, :])
                      + yv[None, None, :, :])

    out = pl.pallas_call(
        body,
        out_shape=jax.ShapeDtypeStruct((N_ACT, N_GRID, N_GRID, D),
                                       jnp.float32),
    )(a_tab, x_tab, y_tab)
    return out.reshape(N_ACT * N_GRID * N_GRID, D)


def _sc_lookup(at_idx, x_idx, y_idx, axy):
    nrows = at_idx.shape[0]
    info = plsc.get_sparse_core_info()
    nw = info.num_cores * info.num_subcores
    assert nrows % (nw * CHUNK) == 0, (nrows, nw)
    rows_per_w = nrows // nw
    steps = rows_per_w // CHUNK

    mesh = plsc.VectorSubcoreMesh(core_axis_name="c", subcore_axis_name="s")

    @functools.partial(
        pl.kernel,
        out_type=jax.ShapeDtypeStruct((nrows * ROW, D), jnp.float32),
        mesh=mesh,
        scratch_types=[
            pltpu.VMEM((CHUNK, ROW), jnp.int32),
            pltpu.VMEM((CHUNK, ROW), jnp.int32),
            pltpu.VMEM((CHUNK, ROW), jnp.int32),
            pltpu.VMEM((CHUNK, ROW), jnp.int32),
            pltpu.VMEM((2, ROW, D), jnp.float32),
            pltpu.SemaphoreType.DMA,
            pltpu.SemaphoreType.DMA,
        ],
    )
    def k(at_hbm, x_hbm, y_hbm, axy_hbm, out_hbm,
          ai_v, xi_v, yi_v, idx_v, rows_v, sem_g, sem_w):
        wid = lax.axis_index("s") * info.num_cores + lax.axis_index("c")
        row_base = wid * rows_per_w

        def _wait_write():
            pltpu.make_async_copy(
                rows_v.at[0], out_hbm.at[pl.ds(0, ROW)], sem_w).wait()

        @pl.loop(0, steps)
        def _step(i):
            row0 = row_base + i * CHUNK
            pltpu.sync_copy(at_hbm.at[pl.ds(row0, CHUNK)], ai_v)
            pltpu.sync_copy(x_hbm.at[pl.ds(row0, CHUNK)], xi_v)
            pltpu.sync_copy(y_hbm.at[pl.ds(row0, CHUNK)], yi_v)
            for j in range(CHUNK):
                for s in range(ROW // 16):
                    sl = pl.ds(s * 16, 16)
                    a = ai_v[j, sl]
                    xx = xi_v[j, sl]
                    yy = yi_v[j, sl]
                    idx_v[j, sl] = (a << 12) | (xx << 6) | yy
            for j in range(CHUNK):
                b = j % 2
                if j >= 2:
                    _wait_write()
                else:
                    pl.when(i > 0)(_wait_write)
                pltpu.async_copy(
                    axy_hbm.at[idx_v.at[j]], rows_v.at[b], sem_g).wait()
                pltpu.async_copy(
                    rows_v.at[b],
                    out_hbm.at[pl.ds((row0 + j) * ROW, ROW)], sem_w)

        _wait_write()
        _wait_write()

    return k(at_idx, x_idx, y_idx, axy)


def kernel(action_type, x, y, action_type_table, x_table, y_table):
    B, T = action_type.shape
    n = B * T
    assert n % ROW == 0
    axy = _build_fused_table(action_type_table, x_table, y_table)
    at2 = action_type.reshape(n // ROW, ROW).astype(jnp.int32)
    x2 = x.reshape(n // ROW, ROW).astype(jnp.int32)
    y2 = y.reshape(n // ROW, ROW).astype(jnp.int32)
    out = _sc_lookup(at2, x2, y2, axy)
    return out.reshape(B, T, D)


# speedup vs baseline: 29.4770x; 1.3340x over previous
import functools

import jax
import jax.numpy as jnp
from jax import lax
from jax.experimental import pallas as pl
from jax.experimental.pallas import tpu as pltpu
from jax.experimental.pallas import tpu_sc as plsc

D = 128
N_ACT = 9
N_GRID = 64
ROW = 128
CHUNK = 32
NBUF = 4


def _build_fused_table(a_tab, x_tab, y_tab):
    def body(a_ref, x_ref, y_ref, o_ref):
        av = a_ref[...]
        xv = x_ref[...]
        yv = y_ref[...]
        o_ref[...] = ((av[:, None, None, :] + xv[None, :, None, :])
                      + yv[None, None, :, :])

    out = pl.pallas_call(
        body,
        out_shape=jax.ShapeDtypeStruct((N_ACT, N_GRID, N_GRID, D),
                                       jnp.float32),
    )(a_tab, x_tab, y_tab)
    return out.reshape(N_ACT * N_GRID * N_GRID, D)


def _fuse_indices(at2, x2, y2):
    rows = at2.shape[0]
    blk = rows // 8

    def body(a_ref, x_ref, y_ref, o_ref):
        o_ref[...] = (a_ref[...] << 12) | (x_ref[...] << 6) | y_ref[...]

    spec = pl.BlockSpec((blk, ROW), lambda i: (i, 0))
    return pl.pallas_call(
        body,
        grid=(8,),
        in_specs=[spec, spec, spec],
        out_specs=spec,
        out_shape=jax.ShapeDtypeStruct((rows, ROW), jnp.int32),
    )(at2, x2, y2)


def _sc_lookup(fused_idx, axy):
    nrows = fused_idx.shape[0]
    info = plsc.get_sparse_core_info()
    nw = info.num_cores * info.num_subcores
    assert nrows % (nw * CHUNK) == 0, (nrows, nw)
    rows_per_w = nrows // nw
    steps = rows_per_w // CHUNK

    mesh = plsc.VectorSubcoreMesh(core_axis_name="c", subcore_axis_name="s")

    @functools.partial(
        pl.kernel,
        out_type=jax.ShapeDtypeStruct((nrows * ROW, D), jnp.float32),
        mesh=mesh,
        scratch_types=[
            pltpu.VMEM((CHUNK, ROW), jnp.int32),
            pltpu.VMEM((NBUF, ROW, D), jnp.float32),
            pltpu.SemaphoreType.DMA,
            pltpu.SemaphoreType.DMA,
        ],
    )
    def k(idx_hbm, axy_hbm, out_hbm, idx_v, rows_v, sem_g, sem_w):
        wid = lax.axis_index("s") * info.num_cores + lax.axis_index("c")
        row_base = wid * rows_per_w

        def _wait_write():
            pltpu.make_async_copy(
                rows_v.at[0], out_hbm.at[pl.ds(0, ROW)], sem_w).wait()

        def _wait_gather():
            pltpu.make_async_copy(
                axy_hbm.at[pl.ds(0, ROW)], rows_v.at[0], sem_g).wait()

        @pl.loop(0, steps)
        def _step(i):
            row0 = row_base + i * CHUNK

            @pl.when(i > 0)
            def _boundary():
                _wait_gather()
                pltpu.async_copy(
                    rows_v.at[(CHUNK - 1) % NBUF],
                    out_hbm.at[pl.ds((row0 - 1) * ROW, ROW)], sem_w)

            pltpu.sync_copy(idx_hbm.at[pl.ds(row0, CHUNK)], idx_v)

            for j in range(CHUNK):
                b = j % NBUF
                if j >= NBUF:
                    _wait_write()
                else:
                    pl.when(i > 0)(_wait_write)
                pltpu.async_copy(
                    axy_hbm.at[idx_v.at[j]], rows_v.at[b], sem_g)
                if j >= 1:
                    _wait_gather()
                    pltpu.async_copy(
                        rows_v.at[(j - 1) % NBUF],
                        out_hbm.at[pl.ds((row0 + j - 1) * ROW, ROW)], sem_w)

        _wait_gather()
        last_row = row_base + rows_per_w - 1
        pltpu.async_copy(
            rows_v.at[(CHUNK - 1) % NBUF],
            out_hbm.at[pl.ds(last_row * ROW, ROW)], sem_w)
        for _ in range(NBUF):
            _wait_write()

    return k(fused_idx, axy)


def kernel(action_type, x, y, action_type_table, x_table, y_table):
    B, T = action_type.shape
    n = B * T
    assert n % ROW == 0
    axy = _build_fused_table(action_type_table, x_table, y_table)
    at2 = action_type.reshape(n // ROW, ROW).astype(jnp.int32)
    x2 = x.reshape(n // ROW, ROW).astype(jnp.int32)
    y2 = y.reshape(n // ROW, ROW).astype(jnp.int32)
    fused = _fuse_indices(at2, x2, y2)
    out = _sc_lookup(fused, axy)
    return out.reshape(B, T, D)


# speedup vs baseline: 29.6499x; 1.0059x over previous
import functools

import jax
import jax.numpy as jnp
from jax import lax
from jax.experimental import pallas as pl
from jax.experimental.pallas import tpu as pltpu
from jax.experimental.pallas import tpu_sc as plsc

D = 128
N_ACT = 9
N_GRID = 64
ROW = 128
CHUNK = 160
BLK = 20
NBUF = 5
LAG = 2


def _build_fused_table(a_tab, x_tab, y_tab):
    def body(a_ref, x_ref, y_ref, o_ref):
        av = a_ref[...]
        xv = x_ref[...]
        yv = y_ref[...]
        o_ref[...] = ((av[:, None, None, :] + xv[None, :, None, :])
                      + yv[None, None, :, :])

    out = pl.pallas_call(
        body,
        out_shape=jax.ShapeDtypeStruct((N_ACT, N_GRID, N_GRID, D),
                                       jnp.float32),
    )(a_tab, x_tab, y_tab)
    return out.reshape(N_ACT * N_GRID * N_GRID, D)


def _fuse_indices(at2, x2, y2):
    rows = at2.shape[0]
    blk = rows // 8

    def body(a_ref, x_ref, y_ref, o_ref):
        o_ref[...] = (a_ref[...] << 12) | (x_ref[...] << 6) | y_ref[...]

    spec = pl.BlockSpec((blk, ROW), lambda i: (i, 0))
    return pl.pallas_call(
        body,
        grid=(8,),
        in_specs=[spec, spec, spec],
        out_specs=spec,
        out_shape=jax.ShapeDtypeStruct((rows, ROW), jnp.int32),
    )(at2, x2, y2)


def _sc_lookup(fused_idx, axy):
    nrows = fused_idx.shape[0]
    info = plsc.get_sparse_core_info()
    nw = info.num_cores * info.num_subcores
    assert nrows % (nw * CHUNK) == 0, (nrows, nw)
    assert CHUNK % BLK == 0 and BLK % NBUF == 0 and LAG < NBUF
    rows_per_w = nrows // nw
    steps = rows_per_w // CHUNK

    mesh = plsc.VectorSubcoreMesh(core_axis_name="c", subcore_axis_name="s")

    @functools.partial(
        pl.kernel,
        out_type=jax.ShapeDtypeStruct((nrows * ROW, D), jnp.float32),
        mesh=mesh,
        scratch_types=[
            pltpu.VMEM((CHUNK, ROW), jnp.int32),
            pltpu.VMEM((NBUF, ROW, D), jnp.float32),
            pltpu.SemaphoreType.DMA,
            pltpu.SemaphoreType.DMA,
        ],
    )
    def k(idx_hbm, axy_hbm, out_hbm, idx_v, rows_v, sem_g, sem_w):
        wid = lax.axis_index("s") * info.num_cores + lax.axis_index("c")
        row_base = wid * rows_per_w

        def _wait_write():
            pltpu.make_async_copy(
                rows_v.at[0], out_hbm.at[pl.ds(0, ROW)], sem_w).wait()

        def _wait_gather():
            pltpu.make_async_copy(
                axy_hbm.at[pl.ds(0, ROW)], rows_v.at[0], sem_g).wait()

        @pl.loop(0, steps)
        def _step(i):
            row0 = row_base + i * CHUNK

            @pl.when(i > 0)
            def _boundary():
                for t in range(LAG):
                    _wait_gather()
                    pltpu.async_copy(
                        rows_v.at[(CHUNK - LAG + t) % NBUF],
                        out_hbm.at[pl.ds((row0 - LAG + t) * ROW, ROW)],
                        sem_w)

            pltpu.sync_copy(idx_hbm.at[pl.ds(row0, CHUNK)], idx_v)

            @pl.loop(0, CHUNK // BLK)
            def _block(m):
                grow = row0 + m * BLK
                for j in range(BLK):
                    if j >= NBUF:
                        _wait_write()
                    else:
                        pl.when((i > 0) | (m > 0))(_wait_write)
                    pltpu.async_copy(
                        axy_hbm.at[idx_v.at[m * BLK + j]],
                        rows_v.at[j % NBUF], sem_g)
                    def _retire(j=j):
                        _wait_gather()
                        pltpu.async_copy(
                            rows_v.at[(j - LAG) % NBUF],
                            out_hbm.at[pl.ds((grow + j - LAG) * ROW, ROW)],
                            sem_w)
                    if j >= LAG:
                        _retire()
                    else:
                        pl.when(m > 0)(_retire)

        for t in range(LAG):
            _wait_gather()
            last_row = row_base + rows_per_w - LAG + t
            pltpu.async_copy(
                rows_v.at[(CHUNK - LAG + t) % NBUF],
                out_hbm.at[pl.ds(last_row * ROW, ROW)], sem_w)
        for _ in range(NBUF):
            _wait_write()

    return k(fused_idx, axy)


def kernel(action_type, x, y, action_type_table, x_table, y_table):
    B, T = action_type.shape
    n = B * T
    assert n % ROW == 0
    axy = _build_fused_table(action_type_table, x_table, y_table)
    at2 = action_type.reshape(n // ROW, ROW).astype(jnp.int32)
    x2 = x.reshape(n // ROW, ROW).astype(jnp.int32)
    y2 = y.reshape(n // ROW, ROW).astype(jnp.int32)
    fused = _fuse_indices(at2, x2, y2)
    out = _sc_lookup(fused, axy)
    return out.reshape(B, T, D)


# speedup vs baseline: 30.1167x; 1.0157x over previous
import functools

import jax
import jax.numpy as jnp
from jax import lax
from jax.experimental import pallas as pl
from jax.experimental.pallas import tpu as pltpu
from jax.experimental.pallas import tpu_sc as plsc

D = 128
N_ACT = 9
N_GRID = 64
ROW = 128
CHUNK = 160
BLK = 20
NBUF = 5
LAG = 3


def _build_fused_table(a_tab, x_tab, y_tab):
    def body(a_ref, x_ref, y_ref, o_ref):
        av = a_ref[...]
        xv = x_ref[...]
        yv = y_ref[...]
        o_ref[...] = ((av[:, None, None, :] + xv[None, :, None, :])
                      + yv[None, None, :, :])

    out = pl.pallas_call(
        body,
        out_shape=jax.ShapeDtypeStruct((N_ACT, N_GRID, N_GRID, D),
                                       jnp.float32),
    )(a_tab, x_tab, y_tab)
    return out.reshape(N_ACT * N_GRID * N_GRID, D)


def _fuse_indices(at, x, y):
    b, t = at.shape
    blk = b // 8

    def body(a_ref, x_ref, y_ref, o_ref):
        o_ref[...] = (a_ref[...] << 12) | (x_ref[...] << 6) | y_ref[...]

    spec = pl.BlockSpec((blk, t), lambda i: (i, 0))
    return pl.pallas_call(
        body,
        grid=(8,),
        in_specs=[spec, spec, spec],
        out_specs=spec,
        out_shape=jax.ShapeDtypeStruct((b, t), jnp.int32),
    )(at, x, y)


def _sc_lookup(fused_idx, axy):
    nrows = fused_idx.shape[0] // ROW
    info = plsc.get_sparse_core_info()
    nw = info.num_cores * info.num_subcores
    assert nrows % (nw * CHUNK) == 0, (nrows, nw)
    assert CHUNK % BLK == 0 and BLK % NBUF == 0 and LAG < NBUF
    rows_per_w = nrows // nw
    steps = rows_per_w // CHUNK

    mesh = plsc.VectorSubcoreMesh(core_axis_name="c", subcore_axis_name="s")

    @functools.partial(
        pl.kernel,
        out_type=jax.ShapeDtypeStruct((nrows * ROW, D), jnp.float32),
        mesh=mesh,
        scratch_types=[
            pltpu.VMEM((CHUNK * ROW,), jnp.int32),
            pltpu.VMEM((NBUF, ROW, D), jnp.float32),
            pltpu.SemaphoreType.DMA,
            pltpu.SemaphoreType.DMA,
        ],
    )
    def k(idx_hbm, axy_hbm, out_hbm, idx_v, rows_v, sem_g, sem_w):
        wid = lax.axis_index("s") * info.num_cores + lax.axis_index("c")
        row_base = wid * rows_per_w

        def _wait_write():
            pltpu.make_async_copy(
                rows_v.at[0], out_hbm.at[pl.ds(0, ROW)], sem_w).wait()

        def _wait_gather():
            pltpu.make_async_copy(
                axy_hbm.at[pl.ds(0, ROW)], rows_v.at[0], sem_g).wait()

        @pl.loop(0, steps)
        def _step(i):
            row0 = row_base + i * CHUNK

            @pl.when(i > 0)
            def _boundary():
                for t in range(LAG):
                    _wait_gather()
                    pltpu.async_copy(
                        rows_v.at[(CHUNK - LAG + t) % NBUF],
                        out_hbm.at[pl.ds((row0 - LAG + t) * ROW, ROW)],
                        sem_w)

            pltpu.sync_copy(
                idx_hbm.at[pl.ds(row0 * ROW, CHUNK * ROW)], idx_v)

            @pl.loop(0, CHUNK // BLK)
            def _block(m):
                grow = row0 + m * BLK
                for j in range(BLK):
                    if j >= NBUF:
                        _wait_write()
                    else:
                        pl.when((i > 0) | (m > 0))(_wait_write)
                    pltpu.async_copy(
                        axy_hbm.at[idx_v.at[pl.ds((m * BLK + j) * ROW, ROW)]],
                        rows_v.at[j % NBUF], sem_g)
                    def _retire(j=j):
                        _wait_gather()
                        pltpu.async_copy(
                            rows_v.at[(j - LAG) % NBUF],
                            out_hbm.at[pl.ds((grow + j - LAG) * ROW, ROW)],
                            sem_w)
                    if j >= LAG:
                        _retire()
                    else:
                        pl.when(m > 0)(_retire)

        for t in range(LAG):
            _wait_gather()
            last_row = row_base + rows_per_w - LAG + t
            pltpu.async_copy(
                rows_v.at[(CHUNK - LAG + t) % NBUF],
                out_hbm.at[pl.ds(last_row * ROW, ROW)], sem_w)
        for _ in range(NBUF):
            _wait_write()

    return k(fused_idx, axy)


def kernel(action_type, x, y, action_type_table, x_table, y_table):
    B, T = action_type.shape
    n = B * T
    assert n % ROW == 0
    axy = _build_fused_table(action_type_table, x_table, y_table)
    fused = _fuse_indices(action_type.astype(jnp.int32),
                          x.astype(jnp.int32),
                          y.astype(jnp.int32)).reshape(n)
    out = _sc_lookup(fused, axy)
    return out.reshape(B, T, D)


# speedup vs baseline: 30.4230x; 1.0102x over previous
import functools

import jax
import jax.numpy as jnp
from jax import lax
from jax.experimental import pallas as pl
from jax.experimental.pallas import tpu as pltpu
from jax.experimental.pallas import tpu_sc as plsc

D = 128
N_ACT = 9
N_GRID = 64
ROW = 128
CHUNK = 160
BLK = 20
NBUF = 5
LAG = 3


def _build_tables_and_fuse(a_tab, x_tab, y_tab, at, x, y):
    b, t = at.shape
    blk = b // 8

    def body(a_ref, x_ref, y_ref, ai_ref, xi_ref, yi_ref, tab_ref, idx_ref):
        @pl.when(pl.program_id(0) == 0)
        def _tab():
            av = a_ref[...]
            xv = x_ref[...]
            yv = y_ref[...]
            tab_ref[...] = ((av[:, None, None, :] + xv[None, :, None, :])
                            + yv[None, None, :, :])

        idx_ref[...] = ((ai_ref[...] << 12) | (xi_ref[...] << 6)
                        | yi_ref[...])

    tspec = pl.BlockSpec((N_ACT, D), lambda i: (0, 0))
    gspec = pl.BlockSpec((N_GRID, D), lambda i: (0, 0))
    ispec = pl.BlockSpec((blk, t), lambda i: (i, 0))
    tab, idx = pl.pallas_call(
        body,
        grid=(8,),
        in_specs=[tspec, gspec, gspec, ispec, ispec, ispec],
        out_specs=[
            pl.BlockSpec((N_ACT, N_GRID, N_GRID, D), lambda i: (0, 0, 0, 0)),
            ispec,
        ],
        out_shape=[
            jax.ShapeDtypeStruct((N_ACT, N_GRID, N_GRID, D), jnp.float32),
            jax.ShapeDtypeStruct((b, t), jnp.int32),
        ],
    )(a_tab, x_tab, y_tab, at, x, y)
    return tab.reshape(N_ACT * N_GRID * N_GRID, D), idx


def _sc_lookup(fused_idx, axy):
    nrows = fused_idx.shape[0] // ROW
    info = plsc.get_sparse_core_info()
    nw = info.num_cores * info.num_subcores
    assert nrows % (nw * CHUNK) == 0, (nrows, nw)
    assert CHUNK % BLK == 0 and BLK % NBUF == 0 and LAG < NBUF
    rows_per_w = nrows // nw
    steps = rows_per_w // CHUNK

    mesh = plsc.VectorSubcoreMesh(core_axis_name="c", subcore_axis_name="s")

    @functools.partial(
        pl.kernel,
        out_type=jax.ShapeDtypeStruct((nrows * ROW, D), jnp.float32),
        mesh=mesh,
        scratch_types=[
            pltpu.VMEM((CHUNK * ROW,), jnp.int32),
            pltpu.VMEM((NBUF, ROW, D), jnp.float32),
            pltpu.SemaphoreType.DMA,
            pltpu.SemaphoreType.DMA,
        ],
    )
    def k(idx_hbm, axy_hbm, out_hbm, idx_v, rows_v, sem_g, sem_w):
        wid = lax.axis_index("s") * info.num_cores + lax.axis_index("c")
        row_base = wid * rows_per_w

        def _wait_write():
            pltpu.make_async_copy(
                rows_v.at[0], out_hbm.at[pl.ds(0, ROW)], sem_w).wait()

        def _wait_gather():
            pltpu.make_async_copy(
                axy_hbm.at[pl.ds(0, ROW)], rows_v.at[0], sem_g).wait()

        @pl.loop(0, steps)
        def _step(i):
            row0 = row_base + i * CHUNK

            @pl.when(i > 0)
            def _boundary():
                for t in range(LAG):
                    _wait_gather()
                    pltpu.async_copy(
                        rows_v.at[(CHUNK - LAG + t) % NBUF],
                        out_hbm.at[pl.ds((row0 - LAG + t) * ROW, ROW)],
                        sem_w)

            pltpu.sync_copy(
                idx_hbm.at[pl.ds(row0 * ROW, CHUNK * ROW)], idx_v)

            @pl.loop(0, CHUNK // BLK)
            def _block(m):
                grow = row0 + m * BLK
                for j in range(BLK):
                    if j >= NBUF:
                        _wait_write()
                    else:
                        pl.when((i > 0) | (m > 0))(_wait_write)
                    pltpu.async_copy(
                        axy_hbm.at[idx_v.at[pl.ds((m * BLK + j) * ROW, ROW)]],
                        rows_v.at[j % NBUF], sem_g)
                    def _retire(j=j):
                        _wait_gather()
                        pltpu.async_copy(
                            rows_v.at[(j - LAG) % NBUF],
                            out_hbm.at[pl.ds((grow + j - LAG) * ROW, ROW)],
                            sem_w)
                    if j >= LAG:
                        _retire()
                    else:
                        pl.when(m > 0)(_retire)

        for t in range(LAG):
            _wait_gather()
            last_row = row_base + rows_per_w - LAG + t
            pltpu.async_copy(
                rows_v.at[(CHUNK - LAG + t) % NBUF],
                out_hbm.at[pl.ds(last_row * ROW, ROW)], sem_w)
        for _ in range(NBUF):
            _wait_write()

    return k(fused_idx, axy)


def kernel(action_type, x, y, action_type_table, x_table, y_table):
    B, T = action_type.shape
    n = B * T
    assert n % ROW == 0
    axy, fused = _build_tables_and_fuse(
        action_type_table, x_table, y_table,
        action_type.astype(jnp.int32), x.astype(jnp.int32),
        y.astype(jnp.int32))
    out = _sc_lookup(fused.reshape(n), axy)
    return out.reshape(B, T, D)
